# baseline (speedup 1.0000x reference)
import functools

import jax
import jax.numpy as jnp
from jax import lax
from jax.experimental import pallas as pl
from jax.experimental.pallas import tpu as pltpu
from jax.experimental.pallas import tpu_sc as plsc

NIN = 10
NOUT = 128
NDISC = 3
NHID = NDISC ** NIN
BATCH = 1024

NC = 2
NS = 16
NW = NC * NS
B_PER_W = BATCH // NW
L = 16

_POW3 = [NDISC ** i for i in range(NIN)]

_mesh = plsc.VectorSubcoreMesh(
    core_axis_name="c", subcore_axis_name="s", num_cores=NC, num_subcores=NS
)


@functools.partial(
    pl.kernel,
    out_type=jax.ShapeDtypeStruct((BATCH, NOUT), jnp.float32),
    mesh=_mesh,
    scratch_types=[
        pltpu.VMEM((NIN * B_PER_W,), jnp.float32),
        pltpu.VMEM((B_PER_W,), jnp.int32),
        pltpu.VMEM((B_PER_W, NOUT), jnp.float32),
        pltpu.SemaphoreType.DMA,
    ],
)
def _sc_lookup(xr_hbm, WT_hbm, out_hbm, xv, idx_v, rows_v, sem):
    wid = lax.axis_index("s") * NC + lax.axis_index("c")
    base = wid * B_PER_W
    pltpu.sync_copy(xr_hbm.at[pl.ds(wid * (NIN * B_PER_W), NIN * B_PER_W)], xv)
    neg = jnp.full((L,), -0.1, jnp.float32)
    pos = jnp.full((L,), 0.1, jnp.float32)
    for ch in range(B_PER_W // L):
        acc = jnp.zeros((L,), jnp.int32)
        for i in range(NIN):
            v = xv[pl.ds(i * B_PER_W + ch * L, L)]
            p3 = jnp.full((L,), _POW3[i], jnp.int32)
            zero = jnp.zeros((L,), jnp.int32)
            d = jnp.where(v > neg, p3, zero) + jnp.where(v > pos, p3, zero)
            acc = acc + d
        idx_v[pl.ds(ch * L, L)] = acc
    pltpu.async_copy(WT_hbm.at[idx_v], rows_v, sem).wait()
    pltpu.sync_copy(rows_v, out_hbm.at[pl.ds(base, B_PER_W)])


def kernel(x, W):
    xr = x.reshape(NW, B_PER_W, NIN).transpose(0, 2, 1).reshape(-1)
    WT = W.T
    return _sc_lookup(xr, WT)
